# baseline (speedup 1.0000x reference)
import jax
import jax.numpy as jnp
from jax import lax
from jax.experimental import pallas as pl
from jax.experimental.pallas import tpu as pltpu
from jax.experimental.pallas import tpu_sc as plsc

_B, _N, _D, _H = 8, 65536, 7, 64
_GH, _GW = 32, 32
_S = _GH * _GW
_C = 4096
_NC = _N // _C

_NCORE, _NSUB = 2, 16
_BN = _B * _N
_W = 128
_HALF_S = (_B // _NCORE) * _S
_PPT = _BN // (_NCORE * _NSUB)
_K = 128
_ROWS_PER_TILE = _HALF_S // _NSUB


def _span_kernel(x_ref, out_ref, spans):
    i = pl.program_id(1)
    xb = x_ref[0]
    colmin = jnp.min(xb, axis=0, keepdims=True)
    colmax = jnp.max(xb, axis=0, keepdims=True)

    @pl.when(i == 0)
    def _init():
        spans[0] = jnp.float32(jnp.inf)
        spans[1] = jnp.float32(-jnp.inf)
        spans[2] = jnp.float32(jnp.inf)
        spans[3] = jnp.float32(-jnp.inf)

    spans[0] = jnp.minimum(spans[0], colmin[0, 0])
    spans[1] = jnp.maximum(spans[1], colmax[0, 0])
    spans[2] = jnp.minimum(spans[2], colmin[0, 1])
    spans[3] = jnp.maximum(spans[3], colmax[0, 1])

    @pl.when(i == _NC - 1)
    def _emit():
        lane = lax.broadcasted_iota(jnp.int32, (1, 128), 1)
        vec = jnp.where(
            lane == 0,
            spans[0],
            jnp.where(lane == 1, spans[1], jnp.where(lane == 2, spans[2], spans[3])),
        )
        out_ref[0] = vec


def _encode_kernel(spans_ref, x_ref, W1_ref, b1_ref, W2_ref, b2_ref,
                   phi_ref, seg_ref):
    b = pl.program_id(0)
    xb = x_ref[0]
    c0 = xb[:, 0:1]
    c1 = xb[:, 1:2]
    x_min = spans_ref[0, 0, 0]
    x_max = spans_ref[0, 0, 1]
    y_min = spans_ref[0, 0, 2]
    y_max = spans_ref[0, 0, 3]
    x_span = jnp.maximum(x_max - x_min, 1e-8)
    y_span = jnp.maximum(y_max - y_min, 1e-8)
    gx = jnp.clip(((c0 - x_min) / x_span * _GH).astype(jnp.int32), 0, _GH - 1)
    gy = jnp.clip(((c1 - y_min) / y_span * _GW).astype(jnp.int32), 0, _GW - 1)
    seg_ref[0, 0] = (
        (gx * _GW + gy + (b % (_B // _NCORE)) * _S).reshape(1, _C)[0]
    )

    h = jnp.maximum(
        jnp.dot(xb, W1_ref[...], preferred_element_type=jnp.float32)
        + b1_ref[...],
        0.0,
    )
    phi = (
        jnp.dot(h, W2_ref[...], preferred_element_type=jnp.float32)
        + b2_ref[...]
    )
    phi_ref[...] = jnp.concatenate(
        [phi, jnp.zeros((_C, _W - _H), jnp.float32)], axis=1
    )


def _sc_segment_sum(phi_hbm, seg_hbm, z_hbm, out_hbm, idx_v, rows_v, acc, sem):
    c = lax.axis_index("c")
    s = lax.axis_index("s")
    pltpu.sync_copy(z_hbm, acc.at[pl.ds(s * _ROWS_PER_TILE, _ROWS_PER_TILE)])
    plsc.subcore_barrier()

    point_base = c * (_BN // _NCORE) + s * _PPT

    def step(j, carry):
        base = point_base + j * _K
        pltpu.sync_copy(seg_hbm.at[pl.ds(base, _K)], idx_v)
        pltpu.async_copy(phi_hbm.at[pl.ds(base, _K), :], rows_v, sem).wait()
        pltpu.sync_copy(rows_v, acc.at[idx_v], add=True)
        return carry

    lax.fori_loop(0, _PPT // _K, step, 0)
    plsc.subcore_barrier()

    src = acc.at[pl.ds(s * _ROWS_PER_TILE, _ROWS_PER_TILE)]
    dst = out_hbm.at[
        pl.ds(c * _HALF_S + s * _ROWS_PER_TILE, _ROWS_PER_TILE)
    ]
    pltpu.sync_copy(src, dst)


def kernel(x, W1, b1, W2, b2):
    spans = pl.pallas_call(
        _span_kernel,
        grid=(_B, _NC),
        in_specs=[pl.BlockSpec((1, _C, _D), lambda b, i: (b, i, 0))],
        out_specs=pl.BlockSpec((1, 1, 128), lambda b, i: (b, 0, 0)),
        out_shape=jax.ShapeDtypeStruct((_B, 1, 128), jnp.float32),
        scratch_shapes=[pltpu.SMEM((4,), jnp.float32)],
    )(x)

    phi, seg = pl.pallas_call(
        _encode_kernel,
        grid=(_B, _NC),
        in_specs=[
            pl.BlockSpec((1, 1, 128), lambda b, i: (b, 0, 0)),
            pl.BlockSpec((1, _C, _D), lambda b, i: (b, i, 0)),
            pl.BlockSpec((_D, _H), lambda b, i: (0, 0)),
            pl.BlockSpec((1, _H), lambda b, i: (0, 0)),
            pl.BlockSpec((_H, _H), lambda b, i: (0, 0)),
            pl.BlockSpec((1, _H), lambda b, i: (0, 0)),
        ],
        out_specs=[
            pl.BlockSpec((_C, _W), lambda b, i: (b * _NC + i, 0)),
            pl.BlockSpec((1, 1, _C), lambda b, i: (b * _NC + i, 0, 0)),
        ],
        out_shape=[
            jax.ShapeDtypeStruct((_BN, _W), jnp.float32),
            jax.ShapeDtypeStruct((_B * _NC, 1, _C), jnp.int32),
        ],
    )(spans, x, W1, b1.reshape(1, _H), W2, b2.reshape(1, _H))

    seg_flat = seg.reshape(_BN)
    zeros = jnp.zeros((_ROWS_PER_TILE, _W), jnp.float32)

    sc_fn = pl.kernel(
        _sc_segment_sum,
        out_type=jax.ShapeDtypeStruct((_B * _S, _W), jnp.float32),
        mesh=plsc.VectorSubcoreMesh(
            core_axis_name="c", subcore_axis_name="s"
        ),
        scratch_types=[
            pltpu.VMEM((_K,), jnp.int32),
            pltpu.VMEM((_K, _W), jnp.float32),
            pltpu.VMEM_SHARED((_HALF_S, _W), jnp.float32),
            pltpu.SemaphoreType.DMA,
        ],
    )
    latent = sc_fn(phi, seg_flat, zeros)
    return latent[:, :_H].reshape(_B, _GH, _GW, _H)


# speedup vs baseline: 1.1872x; 1.1872x over previous
import jax
import jax.numpy as jnp
from jax import lax
from jax.experimental import pallas as pl
from jax.experimental.pallas import tpu as pltpu
from jax.experimental.pallas import tpu_sc as plsc

_B, _N, _D, _H = 8, 65536, 7, 64
_GH, _GW = 32, 32
_S = _GH * _GW
_C = 4096
_NC = _N // _C

_NCORE, _NSUB = 2, 16
_BN = _B * _N
_W = 128
_HALF_S = (_B // _NCORE) * _S
_PPT = _BN // (_NCORE * _NSUB)
_K = 128
_ROWS_PER_TILE = _HALF_S // _NSUB


def _span_kernel(x_ref, out_ref, spans):
    i = pl.program_id(1)
    xb = x_ref[0]
    colmin = jnp.min(xb, axis=0, keepdims=True)
    colmax = jnp.max(xb, axis=0, keepdims=True)

    @pl.when(i == 0)
    def _init():
        spans[0] = jnp.float32(jnp.inf)
        spans[1] = jnp.float32(-jnp.inf)
        spans[2] = jnp.float32(jnp.inf)
        spans[3] = jnp.float32(-jnp.inf)

    spans[0] = jnp.minimum(spans[0], colmin[0, 0])
    spans[1] = jnp.maximum(spans[1], colmax[0, 0])
    spans[2] = jnp.minimum(spans[2], colmin[0, 1])
    spans[3] = jnp.maximum(spans[3], colmax[0, 1])

    @pl.when(i == _NC - 1)
    def _emit():
        lane = lax.broadcasted_iota(jnp.int32, (1, 128), 1)
        vec = jnp.where(
            lane == 0,
            spans[0],
            jnp.where(lane == 1, spans[1], jnp.where(lane == 2, spans[2], spans[3])),
        )
        out_ref[0] = vec


def _encode_kernel(spans_ref, x_ref, W1_ref, b1_ref, W2_ref, b2_ref,
                   phi_ref, seg_ref):
    b = pl.program_id(0)
    xb = x_ref[0]
    c0 = xb[:, 0:1]
    c1 = xb[:, 1:2]
    x_min = spans_ref[0, 0, 0]
    x_max = spans_ref[0, 0, 1]
    y_min = spans_ref[0, 0, 2]
    y_max = spans_ref[0, 0, 3]
    x_span = jnp.maximum(x_max - x_min, 1e-8)
    y_span = jnp.maximum(y_max - y_min, 1e-8)
    gx = jnp.clip(((c0 - x_min) / x_span * _GH).astype(jnp.int32), 0, _GH - 1)
    gy = jnp.clip(((c1 - y_min) / y_span * _GW).astype(jnp.int32), 0, _GW - 1)
    seg_ref[0, 0] = (
        (gx * _GW + gy + (b % (_B // _NCORE)) * _S).reshape(1, _C)[0]
    )

    h = jnp.maximum(
        jnp.dot(xb, W1_ref[...], preferred_element_type=jnp.float32)
        + b1_ref[...],
        0.0,
    )
    phi = (
        jnp.dot(h, W2_ref[...], preferred_element_type=jnp.float32)
        + b2_ref[...]
    )
    phi_ref[...] = jnp.concatenate(
        [phi, jnp.zeros((_C, _W - _H), jnp.float32)], axis=1
    )


def _sc_segment_sum(phi_hbm, seg_hbm, z_hbm, out_hbm,
                    idx_a, idx_b, rows_a, rows_b, acc, sem_a, sem_b):
    c = lax.axis_index("c")
    s = lax.axis_index("s")
    point_base = c * (_BN // _NCORE) + s * _PPT

    def issue(base, idx_v, rows_v, sem):
        pltpu.async_copy(seg_hbm.at[pl.ds(base, _K)], idx_v, sem)
        pltpu.async_copy(phi_hbm.at[pl.ds(base, _K), :], rows_v, sem)

    def drain(base, idx_v, rows_v, sem):
        pltpu.make_async_copy(seg_hbm.at[pl.ds(base, _K)], idx_v, sem).wait()
        pltpu.make_async_copy(
            phi_hbm.at[pl.ds(base, _K), :], rows_v, sem
        ).wait()

    issue(point_base, idx_a, rows_a, sem_a)
    pltpu.sync_copy(z_hbm, acc.at[pl.ds(s * _ROWS_PER_TILE, _ROWS_PER_TILE)])
    plsc.subcore_barrier()

    n_outer = _PPT // _K // 2
    last = point_base + _PPT - _K

    def step(jj, carry):
        base_a = point_base + (2 * jj) * _K
        base_b = base_a + _K
        base_n = jnp.minimum(base_b + _K, last)
        issue(base_b, idx_b, rows_b, sem_b)
        drain(base_a, idx_a, rows_a, sem_a)
        pltpu.sync_copy(rows_a, acc.at[idx_a], add=True)
        issue(base_n, idx_a, rows_a, sem_a)
        drain(base_b, idx_b, rows_b, sem_b)
        pltpu.sync_copy(rows_b, acc.at[idx_b], add=True)
        return carry

    lax.fori_loop(0, n_outer, step, 0)
    drain(last, idx_a, rows_a, sem_a)
    plsc.subcore_barrier()

    src = acc.at[pl.ds(s * _ROWS_PER_TILE, _ROWS_PER_TILE)]
    dst = out_hbm.at[
        pl.ds(c * _HALF_S + s * _ROWS_PER_TILE, _ROWS_PER_TILE)
    ]
    pltpu.sync_copy(src, dst)


def kernel(x, W1, b1, W2, b2):
    spans = pl.pallas_call(
        _span_kernel,
        grid=(_B, _NC),
        in_specs=[pl.BlockSpec((1, _C, _D), lambda b, i: (b, i, 0))],
        out_specs=pl.BlockSpec((1, 1, 128), lambda b, i: (b, 0, 0)),
        out_shape=jax.ShapeDtypeStruct((_B, 1, 128), jnp.float32),
        scratch_shapes=[pltpu.SMEM((4,), jnp.float32)],
    )(x)

    phi, seg = pl.pallas_call(
        _encode_kernel,
        grid=(_B, _NC),
        in_specs=[
            pl.BlockSpec((1, 1, 128), lambda b, i: (b, 0, 0)),
            pl.BlockSpec((1, _C, _D), lambda b, i: (b, i, 0)),
            pl.BlockSpec((_D, _H), lambda b, i: (0, 0)),
            pl.BlockSpec((1, _H), lambda b, i: (0, 0)),
            pl.BlockSpec((_H, _H), lambda b, i: (0, 0)),
            pl.BlockSpec((1, _H), lambda b, i: (0, 0)),
        ],
        out_specs=[
            pl.BlockSpec((_C, _W), lambda b, i: (b * _NC + i, 0)),
            pl.BlockSpec((1, 1, _C), lambda b, i: (b * _NC + i, 0, 0)),
        ],
        out_shape=[
            jax.ShapeDtypeStruct((_BN, _W), jnp.float32),
            jax.ShapeDtypeStruct((_B * _NC, 1, _C), jnp.int32),
        ],
    )(spans, x, W1, b1.reshape(1, _H), W2, b2.reshape(1, _H))

    seg_flat = seg.reshape(_BN)
    zeros = jnp.zeros((_ROWS_PER_TILE, _W), jnp.float32)

    sc_fn = pl.kernel(
        _sc_segment_sum,
        out_type=jax.ShapeDtypeStruct((_B * _S, _W), jnp.float32),
        mesh=plsc.VectorSubcoreMesh(
            core_axis_name="c", subcore_axis_name="s"
        ),
        scratch_types=[
            pltpu.VMEM((_K,), jnp.int32),
            pltpu.VMEM((_K,), jnp.int32),
            pltpu.VMEM((_K, _W), jnp.float32),
            pltpu.VMEM((_K, _W), jnp.float32),
            pltpu.VMEM_SHARED((_HALF_S, _W), jnp.float32),
            pltpu.SemaphoreType.DMA,
            pltpu.SemaphoreType.DMA,
        ],
    )
    latent = sc_fn(phi, seg_flat, zeros)
    return latent[:, :_H].reshape(_B, _GH, _GW, _H)


# speedup vs baseline: 1.8321x; 1.5433x over previous
import jax
import jax.numpy as jnp
from jax import lax
from jax.experimental import pallas as pl
from jax.experimental.pallas import tpu as pltpu
from jax.experimental.pallas import tpu_sc as plsc

_B, _N, _D, _H = 8, 65536, 7, 64
_GH, _GW = 32, 32
_S = _GH * _GW
_C = 4096
_NC = _N // _C

_NCORE, _NSUB = 2, 16
_BN = _B * _N
_W = 128
_HALF_S = (_B // _NCORE) * _S
_PPT = _BN // (_NCORE * _NSUB)
_K = 128
_ROWS_PER_TILE = _HALF_S // _NSUB


def _span_kernel(xc_ref, out_ref, spans):
    i = pl.program_id(1)
    c0 = xc_ref[0, 0:1, :]
    c1 = xc_ref[0, 1:2, :]

    @pl.when(i == 0)
    def _init():
        spans[0] = jnp.float32(jnp.inf)
        spans[1] = jnp.float32(-jnp.inf)
        spans[2] = jnp.float32(jnp.inf)
        spans[3] = jnp.float32(-jnp.inf)

    spans[0] = jnp.minimum(spans[0], jnp.min(c0))
    spans[1] = jnp.maximum(spans[1], jnp.max(c0))
    spans[2] = jnp.minimum(spans[2], jnp.min(c1))
    spans[3] = jnp.maximum(spans[3], jnp.max(c1))

    @pl.when(i == _NC - 1)
    def _emit():
        lane = lax.broadcasted_iota(jnp.int32, (1, 128), 1)
        vec = jnp.where(
            lane == 0,
            spans[0],
            jnp.where(lane == 1, spans[1], jnp.where(lane == 2, spans[2], spans[3])),
        )
        out_ref[0] = vec


def _encode_kernel(spans_ref, x_ref, xc_ref, W1_ref, b1_ref, W2_ref, b2_ref,
                   phi_ref, seg_ref):
    b = pl.program_id(0)
    xb = x_ref[0]
    c0 = xc_ref[0, 0:1, :]
    c1 = xc_ref[0, 1:2, :]
    x_min = spans_ref[0, 0, 0]
    x_max = spans_ref[0, 0, 1]
    y_min = spans_ref[0, 0, 2]
    y_max = spans_ref[0, 0, 3]
    x_span = jnp.maximum(x_max - x_min, 1e-8)
    y_span = jnp.maximum(y_max - y_min, 1e-8)
    gx = jnp.clip(((c0 - x_min) / x_span * _GH).astype(jnp.int32), 0, _GH - 1)
    gy = jnp.clip(((c1 - y_min) / y_span * _GW).astype(jnp.int32), 0, _GW - 1)
    seg_ref[0, 0] = (gx * _GW + gy + (b % (_B // _NCORE)) * _S)[0]

    h = jnp.maximum(
        jnp.dot(xb, W1_ref[...], preferred_element_type=jnp.float32)
        + b1_ref[...],
        0.0,
    )
    phi = (
        jnp.dot(h, W2_ref[...], preferred_element_type=jnp.float32)
        + b2_ref[...]
    )
    phi_ref[...] = jnp.concatenate(
        [phi, jnp.zeros((_C, _W - _H), jnp.float32)], axis=1
    )


def _sc_segment_sum(phi_hbm, seg_hbm, z_hbm, out_hbm,
                    idx_a, idx_b, rows_a, rows_b, acc, sem_a, sem_b):
    c = lax.axis_index("c")
    s = lax.axis_index("s")
    point_base = c * (_BN // _NCORE) + s * _PPT

    def issue(base, idx_v, rows_v, sem):
        pltpu.async_copy(seg_hbm.at[pl.ds(base, _K)], idx_v, sem)
        pltpu.async_copy(phi_hbm.at[pl.ds(base, _K), :], rows_v, sem)

    def drain(base, idx_v, rows_v, sem):
        pltpu.make_async_copy(seg_hbm.at[pl.ds(base, _K)], idx_v, sem).wait()
        pltpu.make_async_copy(
            phi_hbm.at[pl.ds(base, _K), :], rows_v, sem
        ).wait()

    issue(point_base, idx_a, rows_a, sem_a)
    pltpu.sync_copy(z_hbm, acc.at[pl.ds(s * _ROWS_PER_TILE, _ROWS_PER_TILE)])
    plsc.subcore_barrier()

    n_outer = _PPT // _K // 2
    last = point_base + _PPT - _K

    def step(jj, carry):
        base_a = point_base + (2 * jj) * _K
        base_b = base_a + _K
        base_n = jnp.minimum(base_b + _K, last)
        issue(base_b, idx_b, rows_b, sem_b)
        drain(base_a, idx_a, rows_a, sem_a)
        pltpu.sync_copy(rows_a, acc.at[idx_a], add=True)
        issue(base_n, idx_a, rows_a, sem_a)
        drain(base_b, idx_b, rows_b, sem_b)
        pltpu.sync_copy(rows_b, acc.at[idx_b], add=True)
        return carry

    lax.fori_loop(0, n_outer, step, 0)
    drain(last, idx_a, rows_a, sem_a)
    plsc.subcore_barrier()

    src = acc.at[pl.ds(s * _ROWS_PER_TILE, _ROWS_PER_TILE)]
    dst = out_hbm.at[
        pl.ds(c * _HALF_S + s * _ROWS_PER_TILE, _ROWS_PER_TILE)
    ]
    pltpu.sync_copy(src, dst)


def kernel(x, W1, b1, W2, b2):
    xc = jnp.transpose(x[:, :, :2], (0, 2, 1))
    spans = pl.pallas_call(
        _span_kernel,
        grid=(_B, _NC),
        in_specs=[pl.BlockSpec((1, 2, _C), lambda b, i: (b, 0, i))],
        out_specs=pl.BlockSpec((1, 1, 128), lambda b, i: (b, 0, 0)),
        out_shape=jax.ShapeDtypeStruct((_B, 1, 128), jnp.float32),
        scratch_shapes=[pltpu.SMEM((4,), jnp.float32)],
    )(xc)

    phi, seg = pl.pallas_call(
        _encode_kernel,
        grid=(_B, _NC),
        in_specs=[
            pl.BlockSpec((1, 1, 128), lambda b, i: (b, 0, 0)),
            pl.BlockSpec((1, _C, _D), lambda b, i: (b, i, 0)),
            pl.BlockSpec((1, 2, _C), lambda b, i: (b, 0, i)),
            pl.BlockSpec((_D, _H), lambda b, i: (0, 0)),
            pl.BlockSpec((1, _H), lambda b, i: (0, 0)),
            pl.BlockSpec((_H, _H), lambda b, i: (0, 0)),
            pl.BlockSpec((1, _H), lambda b, i: (0, 0)),
        ],
        out_specs=[
            pl.BlockSpec((_C, _W), lambda b, i: (b * _NC + i, 0)),
            pl.BlockSpec((1, 1, _C), lambda b, i: (b * _NC + i, 0, 0)),
        ],
        out_shape=[
            jax.ShapeDtypeStruct((_BN, _W), jnp.float32),
            jax.ShapeDtypeStruct((_B * _NC, 1, _C), jnp.int32),
        ],
    )(spans, x, xc, W1, b1.reshape(1, _H), W2, b2.reshape(1, _H))

    seg_flat = seg.reshape(_BN)
    zeros = jnp.zeros((_ROWS_PER_TILE, _W), jnp.float32)

    sc_fn = pl.kernel(
        _sc_segment_sum,
        out_type=jax.ShapeDtypeStruct((_B * _S, _W), jnp.float32),
        mesh=plsc.VectorSubcoreMesh(
            core_axis_name="c", subcore_axis_name="s"
        ),
        scratch_types=[
            pltpu.VMEM((_K,), jnp.int32),
            pltpu.VMEM((_K,), jnp.int32),
            pltpu.VMEM((_K, _W), jnp.float32),
            pltpu.VMEM((_K, _W), jnp.float32),
            pltpu.VMEM_SHARED((_HALF_S, _W), jnp.float32),
            pltpu.SemaphoreType.DMA,
            pltpu.SemaphoreType.DMA,
        ],
    )
    latent = sc_fn(phi, seg_flat, zeros)
    return latent[:, :_H].reshape(_B, _GH, _GW, _H)


# speedup vs baseline: 1.8736x; 1.0226x over previous
import jax
import jax.numpy as jnp
from jax import lax
from jax.experimental import pallas as pl
from jax.experimental.pallas import tpu as pltpu
from jax.experimental.pallas import tpu_sc as plsc

_B, _N, _D, _H = 8, 65536, 7, 64
_GH, _GW = 32, 32
_S = _GH * _GW
_C = 4096
_NC = _N // _C

_NCORE, _NSUB = 2, 16
_W = 128
_BH = _B // 2
_BNH = _BH * _N
_HALF_S = (_BH // _NCORE) * _S
_PPT = _BNH // (_NCORE * _NSUB)
_K = 128
_ROWS_PER_TILE = _HALF_S // _NSUB


def _span_kernel(xc_ref, out_ref, spans):
    i = pl.program_id(1)
    c0 = xc_ref[0, 0:1, :]
    c1 = xc_ref[0, 1:2, :]

    @pl.when(i == 0)
    def _init():
        spans[0] = jnp.float32(jnp.inf)
        spans[1] = jnp.float32(-jnp.inf)
        spans[2] = jnp.float32(jnp.inf)
        spans[3] = jnp.float32(-jnp.inf)

    spans[0] = jnp.minimum(spans[0], jnp.min(c0))
    spans[1] = jnp.maximum(spans[1], jnp.max(c0))
    spans[2] = jnp.minimum(spans[2], jnp.min(c1))
    spans[3] = jnp.maximum(spans[3], jnp.max(c1))

    @pl.when(i == _NC - 1)
    def _emit():
        lane = lax.broadcasted_iota(jnp.int32, (1, 128), 1)
        vec = jnp.where(
            lane == 0,
            spans[0],
            jnp.where(lane == 1, spans[1], jnp.where(lane == 2, spans[2], spans[3])),
        )
        out_ref[0] = vec


def _encode_kernel(spans_ref, x_ref, xc_ref, W1_ref, b1_ref, W2_ref, b2_ref,
                   phi_ref, seg_ref):
    b = pl.program_id(0)
    xb = x_ref[0]
    c0 = xc_ref[0, 0:1, :]
    c1 = xc_ref[0, 1:2, :]
    x_min = spans_ref[0, 0, 0]
    x_max = spans_ref[0, 0, 1]
    y_min = spans_ref[0, 0, 2]
    y_max = spans_ref[0, 0, 3]
    x_span = jnp.maximum(x_max - x_min, 1e-8)
    y_span = jnp.maximum(y_max - y_min, 1e-8)
    gx = jnp.clip(((c0 - x_min) / x_span * _GH).astype(jnp.int32), 0, _GH - 1)
    gy = jnp.clip(((c1 - y_min) / y_span * _GW).astype(jnp.int32), 0, _GW - 1)
    seg_ref[0, 0] = (gx * _GW + gy + (b % (_BH // _NCORE)) * _S)[0]

    h = jnp.maximum(
        jnp.dot(xb, W1_ref[...], preferred_element_type=jnp.float32)
        + b1_ref[...],
        0.0,
    )
    phi = (
        jnp.dot(h, W2_ref[...], preferred_element_type=jnp.float32)
        + b2_ref[...]
    )
    phi_ref[...] = jnp.concatenate(
        [phi, jnp.zeros((_C, _W - _H), jnp.float32)], axis=1
    )


def _sc_segment_sum(phi_hbm, seg_hbm, z_hbm, out_hbm,
                    idx_a, idx_b, rows_a, rows_b, acc, sem_a, sem_b):
    c = lax.axis_index("c")
    s = lax.axis_index("s")
    point_base = c * (_BNH // _NCORE) + s * _PPT

    def issue(base, idx_v, rows_v, sem):
        pltpu.async_copy(seg_hbm.at[pl.ds(base, _K)], idx_v, sem)
        pltpu.async_copy(phi_hbm.at[pl.ds(base, _K), :], rows_v, sem)

    def drain(base, idx_v, rows_v, sem):
        pltpu.make_async_copy(seg_hbm.at[pl.ds(base, _K)], idx_v, sem).wait()
        pltpu.make_async_copy(
            phi_hbm.at[pl.ds(base, _K), :], rows_v, sem
        ).wait()

    issue(point_base, idx_a, rows_a, sem_a)
    pltpu.sync_copy(z_hbm, acc.at[pl.ds(s * _ROWS_PER_TILE, _ROWS_PER_TILE)])
    plsc.subcore_barrier()

    n_outer = _PPT // _K // 2
    last = point_base + _PPT - _K

    def step(jj, carry):
        base_a = point_base + (2 * jj) * _K
        base_b = base_a + _K
        base_n = jnp.minimum(base_b + _K, last)
        issue(base_b, idx_b, rows_b, sem_b)
        drain(base_a, idx_a, rows_a, sem_a)
        pltpu.sync_copy(rows_a, acc.at[idx_a], add=True)
        issue(base_n, idx_a, rows_a, sem_a)
        drain(base_b, idx_b, rows_b, sem_b)
        pltpu.sync_copy(rows_b, acc.at[idx_b], add=True)
        return carry

    lax.fori_loop(0, n_outer, step, 0)
    drain(last, idx_a, rows_a, sem_a)
    plsc.subcore_barrier()

    src = acc.at[pl.ds(s * _ROWS_PER_TILE, _ROWS_PER_TILE)]
    dst = out_hbm.at[
        pl.ds(c * _HALF_S + s * _ROWS_PER_TILE, _ROWS_PER_TILE)
    ]
    pltpu.sync_copy(src, dst)


def kernel(x, W1, b1, W2, b2):
    xc = jnp.transpose(x[:, :, :2], (0, 2, 1))
    spans = pl.pallas_call(
        _span_kernel,
        grid=(_B, _NC),
        in_specs=[pl.BlockSpec((1, 2, _C), lambda b, i: (b, 0, i))],
        out_specs=pl.BlockSpec((1, 1, 128), lambda b, i: (b, 0, 0)),
        out_shape=jax.ShapeDtypeStruct((_B, 1, 128), jnp.float32),
        scratch_shapes=[pltpu.SMEM((4,), jnp.float32)],
    )(xc)

    def encode(spans_h, x_h, xc_h):
        return pl.pallas_call(
            _encode_kernel,
            grid=(_BH, _NC),
            in_specs=[
                pl.BlockSpec((1, 1, 128), lambda b, i: (b, 0, 0)),
                pl.BlockSpec((1, _C, _D), lambda b, i: (b, i, 0)),
                pl.BlockSpec((1, 2, _C), lambda b, i: (b, 0, i)),
                pl.BlockSpec((_D, _H), lambda b, i: (0, 0)),
                pl.BlockSpec((1, _H), lambda b, i: (0, 0)),
                pl.BlockSpec((_H, _H), lambda b, i: (0, 0)),
                pl.BlockSpec((1, _H), lambda b, i: (0, 0)),
            ],
            out_specs=[
                pl.BlockSpec((_C, _W), lambda b, i: (b * _NC + i, 0)),
                pl.BlockSpec((1, 1, _C), lambda b, i: (b * _NC + i, 0, 0)),
            ],
            out_shape=[
                jax.ShapeDtypeStruct((_BNH, _W), jnp.float32),
                jax.ShapeDtypeStruct((_BH * _NC, 1, _C), jnp.int32),
            ],
        )(spans_h, x_h, xc_h, W1, b1.reshape(1, _H), W2, b2.reshape(1, _H))

    zeros = jnp.zeros((_ROWS_PER_TILE, _W), jnp.float32)
    sc_fn = pl.kernel(
        _sc_segment_sum,
        out_type=jax.ShapeDtypeStruct((_BH * _S, _W), jnp.float32),
        mesh=plsc.VectorSubcoreMesh(
            core_axis_name="c", subcore_axis_name="s"
        ),
        scratch_types=[
            pltpu.VMEM((_K,), jnp.int32),
            pltpu.VMEM((_K,), jnp.int32),
            pltpu.VMEM((_K, _W), jnp.float32),
            pltpu.VMEM((_K, _W), jnp.float32),
            pltpu.VMEM_SHARED((_HALF_S, _W), jnp.float32),
            pltpu.SemaphoreType.DMA,
            pltpu.SemaphoreType.DMA,
        ],
    )

    phi0, seg0 = encode(spans[:_BH], x[:_BH], xc[:_BH])
    phi1, seg1 = encode(spans[_BH:], x[_BH:], xc[_BH:])
    lat0 = sc_fn(phi0, seg0.reshape(_BNH), zeros)
    lat1 = sc_fn(phi1, seg1.reshape(_BNH), zeros)
    latent = jnp.concatenate([lat0, lat1], axis=0)
    return latent[:, :_H].reshape(_B, _GH, _GW, _H)
